# baseline (speedup 1.0000x reference)
import functools

import jax
import jax.numpy as jnp
from jax import lax
from jax.experimental import pallas as pl
from jax.experimental.pallas import tpu as pltpu
from jax.experimental.pallas import tpu_sc as plsc

N = 10000
E = 320000
D = 128
DOUT = 64
K_STEPS = 4

NC = 2
NS = 16
NW = NC * NS
NP = 10240
STRIPE = NP // NS

CH = 64
EPW = 10240
NCHUNK = EPW // CH
EP = EPW * NW

_mesh = plsc.VectorSubcoreMesh(core_axis_name="c", subcore_axis_name="s")


def _fill_zeros_2d(ref, rows):
    z = jnp.zeros((16,), jnp.float32)

    def body(r, _):
        for g in range(8):
            ref[r, pl.ds(g * 16, 16)] = z
        return 0

    lax.fori_loop(0, rows, body, 0)


def _deg_body(dst_hbm, degpart_hbm, dst_v, ones_v, zbuf_v, acc):
    c = lax.axis_index("c")
    s = lax.axis_index("s")
    wid = c * NS + s

    z = jnp.zeros((16,), jnp.float32)

    def zb(i, _):
        zbuf_v[pl.ds(i * 16, 16)] = z
        return 0

    lax.fori_loop(0, STRIPE // 16, zb, 0)
    one = jnp.full((16,), 1.0, jnp.float32)
    for g in range(128 // 16):
        ones_v[pl.ds(g * 16, 16)] = one
    pltpu.sync_copy(zbuf_v, acc.at[pl.ds(s * STRIPE, STRIPE)])
    plsc.subcore_barrier()

    pltpu.sync_copy(dst_hbm.at[wid], dst_v)

    def body(i, _):
        pltpu.sync_copy(ones_v.at[pl.ds(0, CH)], acc.at[dst_v.at[i]], add=True)
        return 0

    lax.fori_loop(0, NCHUNK, body, 0)
    plsc.subcore_barrier()
    pltpu.sync_copy(acc.at[pl.ds(s * STRIPE, STRIPE)],
                    degpart_hbm.at[c, pl.ds(s * STRIPE, STRIPE)])


_deg_call = pl.kernel(
    _deg_body,
    out_type=jax.ShapeDtypeStruct((NC, NP), jnp.float32),
    mesh=_mesh,
    scratch_types=[
        pltpu.VMEM((NCHUNK, CH), jnp.int32),
        pltpu.VMEM((128,), jnp.float32),
        pltpu.VMEM((STRIPE,), jnp.float32),
        pltpu.VMEM_SHARED((NP,), jnp.float32),
    ],
)


NBUF = 5
NROUND = NCHUNK // NBUF
NP2 = NP // 2
OWN = NP // NW
OCH = OWN // CH


def _fused_body(src_hbm, dst_hbm, h0_hbm, dinv_hbm,
                h1_hbm, h2_hbm, h3_hbm, h4_hbm, pdump,
                srcb, dstb, rows, acc, gsems, ssems, isems, gbar):
    c = lax.axis_index("c")
    s = lax.axis_index("s")
    wid = c * NS + s
    base = wid * OWN

    def global_barrier():
        plsc.subcore_barrier()

        @pl.when(c == 0)
        def _():
            pl.semaphore_signal(gbar, 1, core_index=1)

        @pl.when(c == 1)
        def _():
            pl.semaphore_signal(gbar, 1, core_index=0)

        pl.semaphore_wait(gbar, 1)

    def zero_acc_stripe():
        _fill_zeros_2d(rows.at[3], CH)
        for z in range(STRIPE // CH):
            pltpu.sync_copy(rows.at[3],
                            acc.at[pl.ds(s * STRIPE + z * CH, CH), :])

    def ring(h_hbm):
        for b in range(NBUF):
            pltpu.sync_copy(src_hbm.at[wid, b], srcb.at[0, b])
            pltpu.sync_copy(dst_hbm.at[wid, b], dstb.at[0, b])

        def round_body(j, _):
            p = lax.rem(j, 2)
            i0 = j * NBUF
            for b in range(NBUF):
                @pl.when(j > 0)
                def _():
                    pltpu.make_async_copy(
                        rows.at[b], acc.at[dstb.at[0, 0]], ssems.at[b]).wait()
                    pltpu.make_async_copy(
                        src_hbm.at[wid, 0], srcb.at[0, b], isems.at[b]).wait()
                    pltpu.make_async_copy(
                        dst_hbm.at[wid, 0], dstb.at[0, b], isems.at[b]).wait()
                pltpu.async_copy(h_hbm.at[srcb.at[p, b]], rows.at[b],
                                 gsems.at[b])
                @pl.when(j < NROUND - 1)
                def _():
                    pltpu.async_copy(src_hbm.at[wid, i0 + NBUF + b],
                                     srcb.at[1 - p, b], isems.at[b])
                    pltpu.async_copy(dst_hbm.at[wid, i0 + NBUF + b],
                                     dstb.at[1 - p, b], isems.at[b])
            for b in range(NBUF):
                pltpu.make_async_copy(h_hbm.at[srcb.at[p, b]], rows.at[b],
                                      gsems.at[b]).wait()
                pltpu.async_copy(rows.at[b], acc.at[dstb.at[p, b]],
                                 ssems.at[b], add=True)
            return 0

        lax.fori_loop(0, NROUND, round_body, 0)
        for b in range(NBUF):
            pltpu.make_async_copy(rows.at[b], acc.at[dstb.at[0, 0]],
                                  ssems.at[b]).wait()

    def combine(hdst):
        ra, rb, ro, rd = rows.at[0], rows.at[1], rows.at[2], rows.at[3]
        for t in range(OCH):
            pltpu.sync_copy(acc.at[pl.ds(base + t * CH, CH), :], ra)
            pltpu.sync_copy(pdump.at[1 - c, pl.ds(s * OWN + t * CH, CH), :],
                            rb)
            pltpu.sync_copy(dinv_hbm.at[pl.ds(base + t * CH, CH), :], rd)

            def rbody(r, _):
                for g in range(8):
                    sl = pl.ds(g * 16, 16)
                    ro[r, sl] = (ra[r, sl] + rb[r, sl]) * rd[r, sl]
                return 0

            lax.fori_loop(0, CH, rbody, 0)
            pltpu.sync_copy(ro, hdst.at[pl.ds(base + t * CH, CH), :])

    zero_acc_stripe()
    plsc.subcore_barrier()

    houts = [h1_hbm, h2_hbm, h3_hbm, h4_hbm]
    for k in range(K_STEPS):
        hsrc = h0_hbm if k == 0 else houts[k - 1]
        ring(hsrc)
        plsc.subcore_barrier()
        pltpu.sync_copy(acc.at[pl.ds((1 - c) * NP2 + s * OWN, OWN), :],
                        pdump.at[c, pl.ds(s * OWN, OWN), :])
        global_barrier()
        combine(houts[k])
        plsc.subcore_barrier()
        zero_acc_stripe()
        global_barrier()


_fused_call = pl.kernel(
    _fused_body,
    out_type=[jax.ShapeDtypeStruct((NP, D), jnp.float32)] * 4
    + [jax.ShapeDtypeStruct((NC, NP2, D), jnp.float32)],
    mesh=_mesh,
    scratch_types=[
        pltpu.VMEM((2, NBUF, CH), jnp.int32),
        pltpu.VMEM((2, NBUF, CH), jnp.int32),
        pltpu.VMEM((NBUF, CH, D), jnp.float32),
        pltpu.VMEM_SHARED((NP, D), jnp.float32),
        pltpu.SemaphoreType.DMA((NBUF,)),
        pltpu.SemaphoreType.DMA((NBUF,)),
        pltpu.SemaphoreType.DMA((NBUF,)),
        pltpu.SemaphoreType.REGULAR,
    ],
)


RB = 400
GRID = N // RB


def _h0_kern(feats_ref, d0_ref, d1_ref, h0_ref, degc_ref, dinv_ref):
    d = jnp.clip(d0_ref[...] + d1_ref[...], 1.0, None)
    degc_ref[...] = d
    dinv_ref[...] = 1.0 / d
    h0_ref[...] = feats_ref[...] * (0.5 * lax.rsqrt(d))


_h0_call = pl.pallas_call(
    _h0_kern,
    grid=(GRID,),
    in_specs=[
        pl.BlockSpec((RB, D), lambda i: (i, 0)),
        pl.BlockSpec((RB, 1), lambda i: (i, 0)),
        pl.BlockSpec((RB, 1), lambda i: (i, 0)),
    ],
    out_specs=[
        pl.BlockSpec((RB, D), lambda i: (i, 0)),
        pl.BlockSpec((RB, 1), lambda i: (i, 0)),
        pl.BlockSpec((RB, 1), lambda i: (i, 0)),
    ],
    out_shape=[
        jax.ShapeDtypeStruct((N, D), jnp.float32),
        jax.ShapeDtypeStruct((N, 1), jnp.float32),
        jax.ShapeDtypeStruct((N, 1), jnp.float32),
    ],
)


def _final_kern(h0_ref, h1_ref, h2_ref, h3_ref, h4_ref, degc_ref,
                w1_ref, w2_ref, out_ref):
    d = degc_ref[...]
    yh = (h0_ref[...] + h1_ref[...] + h2_ref[...] + h3_ref[...] + h4_ref[...])
    y = yh * (jnp.sqrt(d) * (1.0 / (K_STEPS + 1)))
    hr = jnp.maximum(jnp.dot(y, w1_ref[...],
                             preferred_element_type=jnp.float32), 0.0)
    out_ref[...] = jnp.dot(hr, w2_ref[...],
                           preferred_element_type=jnp.float32)


_final_call = pl.pallas_call(
    _final_kern,
    grid=(GRID,),
    in_specs=[
        pl.BlockSpec((RB, D), lambda i: (i, 0)),
        pl.BlockSpec((RB, D), lambda i: (i, 0)),
        pl.BlockSpec((RB, D), lambda i: (i, 0)),
        pl.BlockSpec((RB, D), lambda i: (i, 0)),
        pl.BlockSpec((RB, D), lambda i: (i, 0)),
        pl.BlockSpec((RB, 1), lambda i: (i, 0)),
        pl.BlockSpec((D, D), lambda i: (0, 0)),
        pl.BlockSpec((D, DOUT), lambda i: (0, 0)),
    ],
    out_specs=pl.BlockSpec((RB, DOUT), lambda i: (i, 0)),
    out_shape=jax.ShapeDtypeStruct((N, DOUT), jnp.float32),
)


def kernel(feats, edge_index, W1, W2):
    src = edge_index[0].astype(jnp.int32)
    dst = edge_index[1].astype(jnp.int32)

    npad = EP - E
    pad_ar = jnp.arange(npad, dtype=jnp.int32)
    src_p = jnp.concatenate([src, pad_ar % N])
    dst_p = jnp.concatenate([dst, N + pad_ar % (NP - N)])
    src3 = src_p.reshape(NW, NCHUNK, CH)
    dst3 = dst_p.reshape(NW, NCHUNK, CH)

    degpart = _deg_call(dst3)
    d0 = degpart[0, :N].reshape(N, 1)
    d1 = degpart[1, :N].reshape(N, 1)
    h0, degc, dinv = _h0_call(feats, d0, d1)
    dinv_pad = jnp.concatenate(
        [dinv[:, 0], jnp.ones((NP - N,), jnp.float32)])
    dinv128 = jnp.broadcast_to(dinv_pad[:, None], (NP, D))

    h1, h2, h3, h4, _ = _fused_call(src3, dst3, h0, dinv128)
    return _final_call(h0, h1, h2, h3, h4, degc, W1, W2)


# speedup vs baseline: 1.0084x; 1.0084x over previous
import jax
import jax.numpy as jnp
from jax import lax
from jax.experimental import pallas as pl
from jax.experimental.pallas import tpu as pltpu
from jax.experimental.pallas import tpu_sc as plsc

N = 10000
E = 320000
D = 128
DOUT = 64
K_STEPS = 4

NC = 2
NS = 16
NW = NC * NS
NP = 10240
STRIPE = NP // NS

CH = 80
EPW = 10240
NCHUNK = EPW // CH
EP = EPW * NW

_mesh = plsc.VectorSubcoreMesh(core_axis_name="c", subcore_axis_name="s")


def _fill_zeros_2d(ref, rows):
    z = jnp.zeros((16,), jnp.float32)

    def body(r, _):
        for g in range(8):
            ref[r, pl.ds(g * 16, 16)] = z
        return 0

    lax.fori_loop(0, rows, body, 0)


def _deg_body(dst_hbm, degpart_hbm, dst_v, ones_v, zbuf_v, acc):
    c = lax.axis_index("c")
    s = lax.axis_index("s")
    wid = c * NS + s

    z = jnp.zeros((16,), jnp.float32)

    def zb(i, _):
        zbuf_v[pl.ds(i * 16, 16)] = z
        return 0

    lax.fori_loop(0, STRIPE // 16, zb, 0)
    one = jnp.full((16,), 1.0, jnp.float32)
    for g in range(128 // 16):
        ones_v[pl.ds(g * 16, 16)] = one
    pltpu.sync_copy(zbuf_v, acc.at[pl.ds(s * STRIPE, STRIPE)])
    plsc.subcore_barrier()

    pltpu.sync_copy(dst_hbm.at[wid], dst_v)

    def body(i, _):
        pltpu.sync_copy(ones_v.at[pl.ds(0, CH)], acc.at[dst_v.at[i]], add=True)
        return 0

    lax.fori_loop(0, NCHUNK, body, 0)
    plsc.subcore_barrier()
    pltpu.sync_copy(acc.at[pl.ds(s * STRIPE, STRIPE)],
                    degpart_hbm.at[c, pl.ds(s * STRIPE, STRIPE)])


_deg_call = pl.kernel(
    _deg_body,
    out_type=jax.ShapeDtypeStruct((NC, NP), jnp.float32),
    mesh=_mesh,
    scratch_types=[
        pltpu.VMEM((NCHUNK, CH), jnp.int32),
        pltpu.VMEM((128,), jnp.float32),
        pltpu.VMEM((STRIPE,), jnp.float32),
        pltpu.VMEM_SHARED((NP,), jnp.float32),
    ],
)


NBUF = 4
NROUND = NCHUNK // NBUF
NP2 = NP // 2
OWN = NP // NW
OCH = OWN // CH


def _fused_body(src_hbm, dst_hbm, h0_hbm, dinv_hbm,
                h1_hbm, h2_hbm, h3_hbm, h4_hbm, pdump,
                srcb, dstb, rows, acc, gsems, ssems, isems, gbar):
    c = lax.axis_index("c")
    s = lax.axis_index("s")
    wid = c * NS + s
    base = wid * OWN

    def global_barrier():
        plsc.subcore_barrier()

        @pl.when(c == 0)
        def _():
            pl.semaphore_signal(gbar, 1, core_index=1)

        @pl.when(c == 1)
        def _():
            pl.semaphore_signal(gbar, 1, core_index=0)

        pl.semaphore_wait(gbar, 1)

    def zero_acc_stripe():
        _fill_zeros_2d(rows.at[3], CH)
        for z in range(STRIPE // CH):
            pltpu.sync_copy(rows.at[3],
                            acc.at[pl.ds(s * STRIPE + z * CH, CH), :])

    def ring(h_hbm):
        for b in range(NBUF):
            pltpu.sync_copy(src_hbm.at[wid, b], srcb.at[0, b])
            pltpu.sync_copy(dst_hbm.at[wid, b], dstb.at[0, b])

        def round_body(j, _):
            p = lax.rem(j, 2)
            i0 = j * NBUF
            for b in range(NBUF):
                @pl.when(j > 0)
                def _():
                    pltpu.make_async_copy(
                        rows.at[b], acc.at[dstb.at[0, 0]], ssems.at[b]).wait()
                    pltpu.make_async_copy(
                        src_hbm.at[wid, 0], srcb.at[0, b], isems.at[b]).wait()
                    pltpu.make_async_copy(
                        dst_hbm.at[wid, 0], dstb.at[0, b], isems.at[b]).wait()
                pltpu.async_copy(h_hbm.at[srcb.at[p, b]], rows.at[b],
                                 gsems.at[b])
                @pl.when(j < NROUND - 1)
                def _():
                    pltpu.async_copy(src_hbm.at[wid, i0 + NBUF + b],
                                     srcb.at[1 - p, b], isems.at[b])
                    pltpu.async_copy(dst_hbm.at[wid, i0 + NBUF + b],
                                     dstb.at[1 - p, b], isems.at[b])
            for b in range(NBUF):
                pltpu.make_async_copy(h_hbm.at[srcb.at[p, b]], rows.at[b],
                                      gsems.at[b]).wait()
                pltpu.async_copy(rows.at[b], acc.at[dstb.at[p, b]],
                                 ssems.at[b], add=True)
            return 0

        lax.fori_loop(0, NROUND, round_body, 0)
        for b in range(NBUF):
            pltpu.make_async_copy(rows.at[b], acc.at[dstb.at[0, 0]],
                                  ssems.at[b]).wait()

    def combine(hdst):
        ra, rb, ro, rd = rows.at[0], rows.at[1], rows.at[2], rows.at[3]
        for t in range(OCH):
            pltpu.sync_copy(acc.at[pl.ds(base + t * CH, CH), :], ra)
            pltpu.sync_copy(pdump.at[1 - c, pl.ds(s * OWN + t * CH, CH), :],
                            rb)
            pltpu.sync_copy(dinv_hbm.at[pl.ds(base + t * CH, CH), :], rd)

            def rbody(r, _):
                for g in range(8):
                    sl = pl.ds(g * 16, 16)
                    ro[r, sl] = (ra[r, sl] + rb[r, sl]) * rd[r, sl]
                return 0

            lax.fori_loop(0, CH, rbody, 0)
            pltpu.sync_copy(ro, hdst.at[pl.ds(base + t * CH, CH), :])

    zero_acc_stripe()
    plsc.subcore_barrier()

    houts = [h1_hbm, h2_hbm, h3_hbm, h4_hbm]
    for k in range(K_STEPS):
        hsrc = h0_hbm if k == 0 else houts[k - 1]
        ring(hsrc)
        plsc.subcore_barrier()
        pltpu.sync_copy(acc.at[pl.ds((1 - c) * NP2 + s * OWN, OWN), :],
                        pdump.at[c, pl.ds(s * OWN, OWN), :])
        global_barrier()
        combine(houts[k])
        plsc.subcore_barrier()
        zero_acc_stripe()
        global_barrier()


_fused_call = pl.kernel(
    _fused_body,
    out_type=[jax.ShapeDtypeStruct((NP, D), jnp.float32)] * 4
    + [jax.ShapeDtypeStruct((NC, NP2, D), jnp.float32)],
    mesh=_mesh,
    scratch_types=[
        pltpu.VMEM((2, NBUF, CH), jnp.int32),
        pltpu.VMEM((2, NBUF, CH), jnp.int32),
        pltpu.VMEM((NBUF, CH, D), jnp.float32),
        pltpu.VMEM_SHARED((NP, D), jnp.float32),
        pltpu.SemaphoreType.DMA((NBUF,)),
        pltpu.SemaphoreType.DMA((NBUF,)),
        pltpu.SemaphoreType.DMA((NBUF,)),
        pltpu.SemaphoreType.REGULAR,
    ],
)


RB = 400
GRID = N // RB


def _h0_kern(feats_ref, d0_ref, d1_ref, h0_ref, degc_ref, dinv_ref):
    d = jnp.clip(d0_ref[...] + d1_ref[...], 1.0, None)
    degc_ref[...] = d
    dinv_ref[...] = 1.0 / d
    h0_ref[...] = feats_ref[...] * (0.5 * lax.rsqrt(d))


_h0_call = pl.pallas_call(
    _h0_kern,
    grid=(GRID,),
    in_specs=[
        pl.BlockSpec((RB, D), lambda i: (i, 0)),
        pl.BlockSpec((RB, 1), lambda i: (i, 0)),
        pl.BlockSpec((RB, 1), lambda i: (i, 0)),
    ],
    out_specs=[
        pl.BlockSpec((RB, D), lambda i: (i, 0)),
        pl.BlockSpec((RB, 1), lambda i: (i, 0)),
        pl.BlockSpec((RB, 1), lambda i: (i, 0)),
    ],
    out_shape=[
        jax.ShapeDtypeStruct((N, D), jnp.float32),
        jax.ShapeDtypeStruct((N, 1), jnp.float32),
        jax.ShapeDtypeStruct((N, 1), jnp.float32),
    ],
)


def _final_kern(h0_ref, h1_ref, h2_ref, h3_ref, h4_ref, degc_ref,
                w1_ref, w2_ref, out_ref):
    d = degc_ref[...]
    yh = (h0_ref[...] + h1_ref[...] + h2_ref[...] + h3_ref[...] + h4_ref[...])
    y = yh * (jnp.sqrt(d) * (1.0 / (K_STEPS + 1)))
    hr = jnp.maximum(jnp.dot(y, w1_ref[...],
                             preferred_element_type=jnp.float32), 0.0)
    out_ref[...] = jnp.dot(hr, w2_ref[...],
                           preferred_element_type=jnp.float32)


_final_call = pl.pallas_call(
    _final_kern,
    grid=(GRID,),
    in_specs=[
        pl.BlockSpec((RB, D), lambda i: (i, 0)),
        pl.BlockSpec((RB, D), lambda i: (i, 0)),
        pl.BlockSpec((RB, D), lambda i: (i, 0)),
        pl.BlockSpec((RB, D), lambda i: (i, 0)),
        pl.BlockSpec((RB, D), lambda i: (i, 0)),
        pl.BlockSpec((RB, 1), lambda i: (i, 0)),
        pl.BlockSpec((D, D), lambda i: (0, 0)),
        pl.BlockSpec((D, DOUT), lambda i: (0, 0)),
    ],
    out_specs=pl.BlockSpec((RB, DOUT), lambda i: (i, 0)),
    out_shape=jax.ShapeDtypeStruct((N, DOUT), jnp.float32),
)


def kernel(feats, edge_index, W1, W2):
    src = edge_index[0].astype(jnp.int32)
    dst = edge_index[1].astype(jnp.int32)

    npad = EP - E
    pad_ar = jnp.arange(npad, dtype=jnp.int32)
    src_p = jnp.concatenate([src, pad_ar % N])
    dst_p = jnp.concatenate([dst, N + pad_ar % (NP - N)])
    src3 = src_p.reshape(NW, NCHUNK, CH)
    dst3 = dst_p.reshape(NW, NCHUNK, CH)

    degpart = _deg_call(dst3)
    d0 = degpart[0, :N].reshape(N, 1)
    d1 = degpart[1, :N].reshape(N, 1)
    h0, degc, dinv = _h0_call(feats, d0, d1)
    dinv_pad = jnp.concatenate(
        [dinv[:, 0], jnp.ones((NP - N,), jnp.float32)])
    dinv128 = jnp.broadcast_to(dinv_pad[:, None], (NP, D))

    h1, h2, h3, h4, _ = _fused_call(src3, dst3, h0, dinv128)
    return _final_call(h0, h1, h2, h3, h4, degc, W1, W2)
